# baseline (speedup 1.0000x reference)
import functools

import jax
import jax.numpy as jnp
from jax import lax
from jax.experimental import pallas as pl
from jax.experimental.pallas import tpu as pltpu
from jax.experimental.pallas import tpu_sc as plsc

BATCH = 4096
SEQ = 200
EMB = 64

NC, NS = 2, 16
NW = NC * NS
BPW = BATCH // NW
NBUF = 4
ET, EI, L = EMB // 8, 8, 128


def _emb_body(idx_hbm, tab_hbm, out_hbm, idx_v, src_v, dst_v, *sems):
    w = lax.axis_index("s") * NC + lax.axis_index("c")
    sem_g = sems[:NBUF]
    sem_o = sems[NBUF:]

    pltpu.sync_copy(idx_hbm.at[w], idx_v)

    def fire_gather(b, s):
        pltpu.async_copy(tab_hbm.at[idx_v.at[s]], src_v.at[b], sem_g[b])

    def drain_gather(b):
        pltpu.make_async_copy(out_hbm.at[0, :, w], dst_v.at[b], sem_g[b]).wait()

    def fire_out(b, s):
        pltpu.async_copy(dst_v.at[b], out_hbm.at[s, :, w], sem_o[b])

    def drain_out(b):
        pltpu.make_async_copy(out_hbm.at[0, :, w], dst_v.at[b], sem_o[b]).wait()

    rows16 = [lax.iota(jnp.int32, 16) + jv * 16 for jv in range(BPW // 16)]

    def transpose(b):
        for e in range(EMB):
            ce = jnp.full((16,), e, jnp.int32)
            et, ei = e // EI, e % EI
            for jv in range(BPW // 16):
                v = plsc.load_gather(src_v.at[b], [rows16[jv], ce])
                dst_v[b, et, ei, pl.ds(jv * 16, 16)] = v

    fire_gather(0, 0)

    @pl.loop(0, SEQ, step=NBUF)
    def _(g):
        for b in range(NBUF):
            s = g + b
            bn = (b + 1) % NBUF

            @pl.when(s + 1 < SEQ)
            def _():
                fire_gather(bn, s + 1)

            drain_gather(b)

            @pl.when(s >= NBUF)
            def _():
                drain_out(b)

            transpose(b)
            fire_out(b, s)

    for b in range(NBUF):
        drain_out(b)


@jax.jit
def kernel(input_tensor, weight):
    idx = (
        input_tensor.reshape(NW, BPW, SEQ).transpose(0, 2, 1).astype(jnp.int32)
    )
    mesh = plsc.VectorSubcoreMesh(
        core_axis_name="c", subcore_axis_name="s", num_cores=NC, num_subcores=NS
    )
    out5 = pl.kernel(
        _emb_body,
        out_type=jax.ShapeDtypeStruct((SEQ, ET, NW, EI, L), jnp.float32),
        mesh=mesh,
        scratch_types=[
            pltpu.VMEM((SEQ, BPW), jnp.int32),
            pltpu.VMEM((NBUF, BPW, EMB), jnp.float32),
            pltpu.VMEM((NBUF, ET, EI, L), jnp.float32),
        ]
        + [pltpu.SemaphoreType.DMA] * (2 * NBUF),
        compiler_params=pltpu.CompilerParams(
            use_tc_tiling_on_sc=False, needs_layout_passes=False
        ),
    )(idx, weight)
    return out5.transpose(2, 4, 0, 1, 3).reshape(BATCH, SEQ, EMB)


# speedup vs baseline: 1.2117x; 1.2117x over previous
import functools

import jax
import jax.numpy as jnp
from jax import lax
from jax.experimental import pallas as pl
from jax.experimental.pallas import tpu as pltpu
from jax.experimental.pallas import tpu_sc as plsc

BATCH = 4096
SEQ = 200
EMB = 64

NC, NS = 2, 16
NW = NC * NS
BPW = BATCH // NW
NBUF = 4
ET, EI, L = EMB // 8, 8, 128


def _emb_body(idx_hbm, tab_hbm, out_hbm, idx_v, src_v, dst_v, *sems):
    w = lax.axis_index("s") * NC + lax.axis_index("c")
    sem_g = sems[:NBUF]
    sem_o = sems[NBUF:]

    pltpu.sync_copy(idx_hbm.at[w], idx_v)

    def fire_gather(b, s):
        pltpu.async_copy(tab_hbm.at[idx_v.at[s]], src_v.at[b], sem_g[b])

    def drain_gather(b):
        pltpu.make_async_copy(out_hbm.at[0, :, w], dst_v.at[b], sem_g[b]).wait()

    def fire_out(b, s):
        pltpu.async_copy(dst_v.at[b], out_hbm.at[s, :, w], sem_o[b])

    def drain_out(b):
        pltpu.make_async_copy(out_hbm.at[0, :, w], dst_v.at[b], sem_o[b]).wait()

    rows16 = [lax.iota(jnp.int32, 16) + jv * 16 for jv in range(BPW // 16)]

    def transpose(b):
        for e0 in range(0, EMB, 2):
            vs = []
            for e in (e0, e0 + 1):
                ce = jnp.full((16,), e, jnp.int32)
                for jv in range(BPW // 16):
                    vs.append(plsc.load_gather(src_v.at[b], [rows16[jv], ce]))
            for i, e in enumerate((e0, e0 + 1)):
                et, ei = e // EI, e % EI
                for jv in range(BPW // 16):
                    dst_v[b, et, ei, pl.ds(jv * 16, 16)] = vs[i * 8 + jv]

    fire_gather(0, 0)

    @pl.loop(0, SEQ, step=NBUF)
    def _(g):
        for b in range(NBUF):
            s = g + b
            bn = (b + 1) % NBUF

            @pl.when(s + 1 < SEQ)
            def _():
                fire_gather(bn, s + 1)

            drain_gather(b)

            @pl.when(s >= NBUF)
            def _():
                drain_out(b)

            transpose(b)
            fire_out(b, s)

    for b in range(NBUF):
        drain_out(b)


@jax.jit
def kernel(input_tensor, weight):
    idx = (
        input_tensor.reshape(NW, BPW, SEQ).transpose(0, 2, 1).astype(jnp.int32)
    )
    mesh = plsc.VectorSubcoreMesh(
        core_axis_name="c", subcore_axis_name="s", num_cores=NC, num_subcores=NS
    )
    out5 = pl.kernel(
        _emb_body,
        out_type=jax.ShapeDtypeStruct((SEQ, ET, NW, EI, L), jnp.float32),
        mesh=mesh,
        scratch_types=[
            pltpu.VMEM((SEQ, BPW), jnp.int32),
            pltpu.VMEM((NBUF, BPW, EMB), jnp.float32),
            pltpu.VMEM((NBUF, ET, EI, L), jnp.float32),
        ]
        + [pltpu.SemaphoreType.DMA] * (2 * NBUF),
        compiler_params=pltpu.CompilerParams(
            use_tc_tiling_on_sc=False, needs_layout_passes=False
        ),
    )(idx, weight)
    return out5.transpose(2, 4, 0, 1, 3).reshape(BATCH, SEQ, EMB)


# speedup vs baseline: 2.2875x; 1.8878x over previous
import functools

import jax
import jax.numpy as jnp
from jax import lax
from jax.experimental import pallas as pl
from jax.experimental.pallas import tpu as pltpu
from jax.experimental.pallas import tpu_sc as plsc

BATCH = 4096
SEQ = 200
EMB = 64

NC, NS = 2, 16
NW = NC * NS
BPW = BATCH // NW
NBUF = 4
ET, EI, L = EMB // 8, 8, 128


def _emb_body(idx_hbm, tab_hbm, out_hbm, idx_v, src_v, dst_v, *sems):
    w = lax.axis_index("s") * NC + lax.axis_index("c")
    sem_g = sems[:NBUF]
    sem_o = sems[NBUF:]

    pltpu.sync_copy(idx_hbm.at[w], idx_v)

    def fire_gather(b, s):
        pltpu.async_copy(tab_hbm.at[idx_v.at[s]], src_v.at[b], sem_g[b])

    def dst_blk(b):
        return dst_v.at[b, :, :, pl.ds(0, L)]

    def drain_gather(b):
        pltpu.make_async_copy(out_hbm.at[0, :, w], dst_blk(b), sem_g[b]).wait()

    def fire_out(b, s):
        pltpu.async_copy(dst_blk(b), out_hbm.at[s, :, w], sem_o[b])

    def drain_out(b):
        pltpu.make_async_copy(out_hbm.at[0, :, w], dst_blk(b), sem_o[b]).wait()

    eidx = [lax.iota(jnp.int32, 16) + g * 16 for g in range(EMB // 16)]
    et16 = [e >> 3 for e in eidx]
    ei16 = [e & 7 for e in eidx]

    def transpose(b):
        for j0 in range(0, BPW, 4):
            vs = []
            for j in range(j0, j0 + 4):
                for g in range(EMB // 16):
                    vs.append(src_v[b, j, pl.ds(g * 16, 16)])
            for i, j in enumerate(range(j0, j0 + 4)):
                cj = jnp.full((16,), j, jnp.int32)
                for g in range(EMB // 16):
                    plsc.store_scatter(
                        dst_v.at[b], [et16[g], ei16[g], cj], vs[i * 4 + g]
                    )

    fire_gather(0, 0)

    @pl.loop(0, SEQ, step=NBUF)
    def _(g):
        for b in range(NBUF):
            s = g + b
            bn = (b + 1) % NBUF

            @pl.when(s + 1 < SEQ)
            def _():
                fire_gather(bn, s + 1)

            drain_gather(b)

            @pl.when(s >= NBUF)
            def _():
                drain_out(b)

            transpose(b)
            fire_out(b, s)

    for b in range(NBUF):
        drain_out(b)


@jax.jit
def kernel(input_tensor, weight):
    idx = (
        input_tensor.reshape(NW, BPW, SEQ).transpose(0, 2, 1).astype(jnp.int32)
    )
    mesh = plsc.VectorSubcoreMesh(
        core_axis_name="c", subcore_axis_name="s", num_cores=NC, num_subcores=NS
    )
    out5 = pl.kernel(
        _emb_body,
        out_type=jax.ShapeDtypeStruct((SEQ, ET, NW, EI, L), jnp.float32),
        mesh=mesh,
        scratch_types=[
            pltpu.VMEM((SEQ, BPW), jnp.int32),
            pltpu.VMEM((NBUF, BPW, EMB), jnp.float32),
            pltpu.VMEM((NBUF, ET, EI, L + 1), jnp.float32),
        ]
        + [pltpu.SemaphoreType.DMA] * (2 * NBUF),
        compiler_params=pltpu.CompilerParams(
            use_tc_tiling_on_sc=False, needs_layout_passes=False
        ),
    )(idx, weight)
    return out5.transpose(2, 4, 0, 1, 3).reshape(BATCH, SEQ, EMB)
